# Initial kernel scaffold; baseline (speedup 1.0000x reference)
import jax
import jax.numpy as jnp
from jax.experimental import pallas as pl


def kernel(features, edge_index, W, b):
    raise NotImplementedError("write your pallas kernel here")


# speedup vs baseline: 3.4216x; 3.4216x over previous
import jax
import jax.numpy as jnp
from jax import lax
from jax.experimental import pallas as pl
from jax.experimental.pallas import tpu as pltpu
from jax.experimental.pallas import tpu_sc as plsc

N = 10000
NP = 10240
E = 160000
D = 256
Q = 64
NC = 2
NS = 16
K = 80
EPT = E // NS
CH = EPT // K
RPT = NP // NS
ZR = 128


def _sc_body(f0, f1, f2, f3, col_h, row_h, s_out, deg_out,
             col_v, row_v, rows_v, zrow_v, degl_v, red_v, degsum_v,
             accum_s, stage_s, sem):
  c = lax.axis_index("c")
  s = lax.axis_index("s")
  zeros16 = jnp.zeros((16,), jnp.float32)
  ones16 = jnp.ones((16,), jnp.float32)

  def fill_zrow(i, _):
    r = i // (Q // 16)
    o = (i % (Q // 16)) * 16
    zrow_v[r, pl.ds(o, 16)] = zeros16
    return 0
  lax.fori_loop(0, ZR * (Q // 16), fill_zrow, 0)

  def fill_degl(i, _):
    degl_v[pl.ds(i * 16, 16)] = zeros16
    return 0
  lax.fori_loop(0, NP // 16, fill_degl, 0)

  pltpu.sync_copy(col_h.at[s], col_v)
  pltpu.sync_copy(row_h.at[s], row_v)

  def zero_accum():
    for j in range(RPT // ZR):
      pltpu.sync_copy(zrow_v, accum_s.at[pl.ds(s * RPT + j * ZR, ZR)])

  def sweep(table, with_deg):
    def chunk(g, _):
      idx = col_v.at[g]
      ridx = row_v.at[g]
      pltpu.async_copy(table.at[idx], rows_v, sem).wait()
      pltpu.sync_copy(rows_v, accum_s.at[ridx], add=True)
      if with_deg:
        for j in range(K // 16):
          idx16 = row_v[g, pl.ds(j * 16, 16)]
          plsc.addupdate_scatter(degl_v, [idx16], ones16)
      return 0
    lax.fori_loop(0, CH, chunk, 0)

  def copy_out(q):
    for j in range(RPT // ZR):
      sl = pl.ds(s * RPT + j * ZR, ZR)
      pltpu.sync_copy(accum_s.at[sl], s_out.at[q].at[sl])

  def both_passes(ta, tb, qa, qb, with_deg):
    zero_accum()
    plsc.subcore_barrier()
    sweep(ta, with_deg)
    plsc.subcore_barrier()
    copy_out(qa)
    plsc.subcore_barrier()
    zero_accum()
    plsc.subcore_barrier()
    sweep(tb, False)
    plsc.subcore_barrier()
    copy_out(qb)

  @pl.when(c == 0)
  def _():
    both_passes(f0, f1, 0, 1, True)

  @pl.when(c == 1)
  def _():
    both_passes(f2, f3, 2, 3, False)

  @pl.when(c == 0)
  def _():
    pltpu.sync_copy(degl_v, stage_s.at[s])
    plsc.subcore_barrier()
    pltpu.sync_copy(stage_s.at[:, pl.ds(s * RPT, RPT)], red_v)

    def redj(j, _):
      acc = red_v[0, pl.ds(j * 16, 16)]
      for t in range(1, NS):
        acc = acc + red_v[t, pl.ds(j * 16, 16)]
      degsum_v[pl.ds(j * 16, 16)] = acc
      return 0
    lax.fori_loop(0, RPT // 16, redj, 0)
    pltpu.sync_copy(degsum_v, deg_out.at[pl.ds(s * RPT, RPT)])


_sc_call = pl.kernel(
    _sc_body,
    out_type=(
        jax.ShapeDtypeStruct((4, NP, Q), jnp.float32),
        jax.ShapeDtypeStruct((NP,), jnp.float32),
    ),
    mesh=plsc.VectorSubcoreMesh(core_axis_name="c", subcore_axis_name="s"),
    compiler_params=pltpu.CompilerParams(
        needs_layout_passes=False, use_tc_tiling_on_sc=False),
    scratch_types=[
        pltpu.VMEM((CH, K), jnp.int32),
        pltpu.VMEM((CH, K), jnp.int32),
        pltpu.VMEM((K, Q), jnp.float32),
        pltpu.VMEM((ZR, Q), jnp.float32),
        pltpu.VMEM((NP,), jnp.float32),
        pltpu.VMEM((NS, RPT), jnp.float32),
        pltpu.VMEM((RPT,), jnp.float32),
        pltpu.VMEM_SHARED((NP, Q), jnp.float32),
        pltpu.VMEM_SHARED((NS, NP), jnp.float32),
        pltpu.SemaphoreType.DMA,
    ],
)


def _tc_body(s0_ref, s1_ref, s2_ref, s3_ref, deg_ref, wt_ref, b_ref, out_ref):
  wt = wt_ref[...]
  a = jnp.dot(s0_ref[0], wt[0 * Q:1 * Q, :], preferred_element_type=jnp.float32)
  a = a + jnp.dot(s1_ref[0], wt[1 * Q:2 * Q, :], preferred_element_type=jnp.float32)
  a = a + jnp.dot(s2_ref[0], wt[2 * Q:3 * Q, :], preferred_element_type=jnp.float32)
  a = a + jnp.dot(s3_ref[0], wt[3 * Q:4 * Q, :], preferred_element_type=jnp.float32)
  a = a + deg_ref[...] * b_ref[...]
  out_ref[...] = jnp.dot(a, wt, preferred_element_type=jnp.float32) + b_ref[...]


_TC_R = 1000


def _tc_call(s4, degm, wt, b2):
  blk = lambda q: pl.BlockSpec((1, _TC_R, Q), lambda i, q=q: (q, i, 0))
  return pl.pallas_call(
      _tc_body,
      grid=(N // _TC_R,),
      in_specs=[
          blk(0), blk(1), blk(2), blk(3),
          pl.BlockSpec((_TC_R, 1), lambda i: (i, 0)),
          pl.BlockSpec((D, D), lambda i: (0, 0)),
          pl.BlockSpec((1, D), lambda i: (0, 0)),
      ],
      out_specs=pl.BlockSpec((_TC_R, D), lambda i: (i, 0)),
      out_shape=jax.ShapeDtypeStruct((N, D), jnp.float32),
  )(s4, s4, s4, s4, degm, wt, b2)


@jax.jit
def kernel(features, edge_index, W, b):
  row = edge_index[0].astype(jnp.int32)
  col = edge_index[1].astype(jnp.int32)
  col_h = col.reshape(NS, CH, K)
  row_h = row.reshape(NS, CH, K)
  fq = [features[:, q * Q:(q + 1) * Q] for q in range(4)]
  s4, deg = _sc_call(fq[0], fq[1], fq[2], fq[3], col_h, row_h)
  degm = deg.reshape(NP, 1)
  wt = W.T
  b2 = b.reshape(1, D)
  return _tc_call(s4, degm, wt, b2)


# speedup vs baseline: 5.1691x; 1.5107x over previous
import jax
import jax.numpy as jnp
from jax import lax
from jax.experimental import pallas as pl
from jax.experimental.pallas import tpu as pltpu
from jax.experimental.pallas import tpu_sc as plsc

N = 10000
NP = 10240
E = 160000
D = 256
Q = 64
NC = 2
NS = 16
K = 80
EPT = E // NS
CH = EPT // K
RPT = NP // NS
ZR = 128


def _sc_body(f0, f1, f2, f3, col_h, row_h, s_out, deg_out,
             col_v, row_v, rows0_v, rows1_v, zrow_v, degl_v, red_v, degsum_v,
             accum_s, stage_s, g0, g1, s0, s1):
  c = lax.axis_index("c")
  s = lax.axis_index("s")
  zeros16 = jnp.zeros((16,), jnp.float32)
  ones16 = jnp.ones((16,), jnp.float32)

  def fill_zrow(i, _):
    r = i // (Q // 16)
    o = (i % (Q // 16)) * 16
    zrow_v[r, pl.ds(o, 16)] = zeros16
    return 0
  lax.fori_loop(0, ZR * (Q // 16), fill_zrow, 0)

  def fill_degl(i, _):
    degl_v[pl.ds(i * 16, 16)] = zeros16
    return 0
  lax.fori_loop(0, NP // 16, fill_degl, 0)

  pltpu.sync_copy(col_h.at[s], col_v)
  pltpu.sync_copy(row_h.at[s], row_v)

  def zero_accum():
    for j in range(RPT // ZR):
      pltpu.sync_copy(zrow_v, accum_s.at[pl.ds(s * RPT + j * ZR, ZR)])

  def sweep(table, with_deg):
    def hist(g):
      if with_deg:
        for j in range(K // 16):
          idx16 = row_v[g, pl.ds(j * 16, 16)]
          plsc.addupdate_scatter(degl_v, [idx16], ones16)

    def stage_pair(g, buf, gsem, ssem, next_g):
      pltpu.make_async_copy(table.at[col_v.at[g]], buf, gsem).wait()
      pltpu.async_copy(buf, accum_s.at[row_v.at[g]], ssem, add=True)
      hist(g)
      pltpu.make_async_copy(buf, accum_s.at[row_v.at[g]], ssem).wait()
      if next_g is not None:
        pltpu.async_copy(table.at[col_v.at[next_g]], buf, gsem)

    pltpu.async_copy(table.at[col_v.at[0]], rows0_v, g0)

    def body(t2, _):
      t = 2 * t2
      pltpu.async_copy(table.at[col_v.at[t + 1]], rows1_v, g1)
      stage_pair(t, rows0_v, g0, s0, t + 2)
      stage_pair(t + 1, rows1_v, g1, s1, None)
      return 0
    lax.fori_loop(0, (CH - 1) // 2, body, 0)
    pltpu.make_async_copy(table.at[col_v.at[CH - 1]], rows0_v, g0).wait()
    pltpu.sync_copy(rows0_v, accum_s.at[row_v.at[CH - 1]], add=True)
    hist(CH - 1)

  def copy_out(q):
    for j in range(RPT // ZR):
      sl = pl.ds(s * RPT + j * ZR, ZR)
      pltpu.sync_copy(accum_s.at[sl], s_out.at[q].at[sl])

  def both_passes(ta, tb, qa, qb, with_deg):
    zero_accum()
    plsc.subcore_barrier()
    sweep(ta, with_deg)
    plsc.subcore_barrier()
    copy_out(qa)
    plsc.subcore_barrier()
    zero_accum()
    plsc.subcore_barrier()
    sweep(tb, False)
    plsc.subcore_barrier()
    copy_out(qb)

  @pl.when(c == 0)
  def _():
    both_passes(f0, f1, 0, 1, True)

  @pl.when(c == 1)
  def _():
    both_passes(f2, f3, 2, 3, False)

  @pl.when(c == 0)
  def _():
    pltpu.sync_copy(degl_v, stage_s.at[s])
    plsc.subcore_barrier()
    pltpu.sync_copy(stage_s.at[:, pl.ds(s * RPT, RPT)], red_v)

    def redj(j, _):
      acc = red_v[0, pl.ds(j * 16, 16)]
      for t in range(1, NS):
        acc = acc + red_v[t, pl.ds(j * 16, 16)]
      degsum_v[pl.ds(j * 16, 16)] = acc
      return 0
    lax.fori_loop(0, RPT // 16, redj, 0)
    pltpu.sync_copy(degsum_v, deg_out.at[pl.ds(s * RPT, RPT)])


_sc_call = pl.kernel(
    _sc_body,
    out_type=(
        jax.ShapeDtypeStruct((4, NP, Q), jnp.float32),
        jax.ShapeDtypeStruct((NP,), jnp.float32),
    ),
    mesh=plsc.VectorSubcoreMesh(core_axis_name="c", subcore_axis_name="s"),
    compiler_params=pltpu.CompilerParams(
        needs_layout_passes=False, use_tc_tiling_on_sc=False),
    scratch_types=[
        pltpu.VMEM((CH, K), jnp.int32),
        pltpu.VMEM((CH, K), jnp.int32),
        pltpu.VMEM((K, Q), jnp.float32),
        pltpu.VMEM((K, Q), jnp.float32),
        pltpu.VMEM((ZR, Q), jnp.float32),
        pltpu.VMEM((NP,), jnp.float32),
        pltpu.VMEM((NS, RPT), jnp.float32),
        pltpu.VMEM((RPT,), jnp.float32),
        pltpu.VMEM_SHARED((NP, Q), jnp.float32),
        pltpu.VMEM_SHARED((NS, NP), jnp.float32),
        pltpu.SemaphoreType.DMA,
        pltpu.SemaphoreType.DMA,
        pltpu.SemaphoreType.DMA,
        pltpu.SemaphoreType.DMA,
    ],
)


def _tc_body(s0_ref, s1_ref, s2_ref, s3_ref, deg_ref, wt_ref, b_ref, out_ref):
  wt = wt_ref[...]
  a = jnp.dot(s0_ref[0], wt[0 * Q:1 * Q, :], preferred_element_type=jnp.float32)
  a = a + jnp.dot(s1_ref[0], wt[1 * Q:2 * Q, :], preferred_element_type=jnp.float32)
  a = a + jnp.dot(s2_ref[0], wt[2 * Q:3 * Q, :], preferred_element_type=jnp.float32)
  a = a + jnp.dot(s3_ref[0], wt[3 * Q:4 * Q, :], preferred_element_type=jnp.float32)
  a = a + deg_ref[...] * b_ref[...]
  out_ref[...] = jnp.dot(a, wt, preferred_element_type=jnp.float32) + b_ref[...]


_TC_R = 1000


def _tc_call(s4, degm, wt, b2):
  blk = lambda q: pl.BlockSpec((1, _TC_R, Q), lambda i, q=q: (q, i, 0))
  return pl.pallas_call(
      _tc_body,
      grid=(N // _TC_R,),
      in_specs=[
          blk(0), blk(1), blk(2), blk(3),
          pl.BlockSpec((_TC_R, 1), lambda i: (i, 0)),
          pl.BlockSpec((D, D), lambda i: (0, 0)),
          pl.BlockSpec((1, D), lambda i: (0, 0)),
      ],
      out_specs=pl.BlockSpec((_TC_R, D), lambda i: (i, 0)),
      out_shape=jax.ShapeDtypeStruct((N, D), jnp.float32),
  )(s4, s4, s4, s4, degm, wt, b2)


@jax.jit
def kernel(features, edge_index, W, b):
  row = edge_index[0].astype(jnp.int32)
  col = edge_index[1].astype(jnp.int32)
  col_h = col.reshape(NS, CH, K)
  row_h = row.reshape(NS, CH, K)
  fq = [features[:, q * Q:(q + 1) * Q] for q in range(4)]
  s4, deg = _sc_call(fq[0], fq[1], fq[2], fq[3], col_h, row_h)
  degm = deg.reshape(NP, 1)
  wt = W.T
  b2 = b.reshape(1, D)
  return _tc_call(s4, degm, wt, b2)
